# Initial kernel scaffold; baseline (speedup 1.0000x reference)
import jax
import jax.numpy as jnp
from jax.experimental import pallas as pl


def kernel(x, pe):
    raise NotImplementedError("write your pallas kernel here")


# speedup vs baseline: 2.8599x; 2.8599x over previous
import jax
import jax.numpy as jnp
from jax.experimental import pallas as pl


def _add_pe_body(x_ref, pe_ref, o_ref):
    o_ref[...] = x_ref[...] + pe_ref[...][None]


def kernel(x, pe):
    batch, seq_len, d_model = x.shape
    block_s = 512
    while seq_len % block_s:
        block_s //= 2
    grid = (seq_len // block_s, batch)
    return pl.pallas_call(
        _add_pe_body,
        grid=grid,
        in_specs=[
            pl.BlockSpec((1, block_s, d_model), lambda s, b: (b, s, 0)),
            pl.BlockSpec((block_s, d_model), lambda s, b: (s, 0)),
        ],
        out_specs=pl.BlockSpec((1, block_s, d_model), lambda s, b: (b, s, 0)),
        out_shape=jax.ShapeDtypeStruct(x.shape, x.dtype),
    )(x, pe)


# speedup vs baseline: 3.1764x; 1.1107x over previous
import jax
import jax.numpy as jnp
from jax.experimental import pallas as pl


def _add_pe_body(x_ref, pe_ref, o_ref):
    o_ref[...] = x_ref[...] + pe_ref[...][None]


def kernel(x, pe):
    batch, seq_len, d_model = x.shape
    block_s = 1024
    while seq_len % block_s:
        block_s //= 2
    grid = (seq_len // block_s, batch)
    return pl.pallas_call(
        _add_pe_body,
        grid=grid,
        in_specs=[
            pl.BlockSpec((1, block_s, d_model), lambda s, b: (b, s, 0)),
            pl.BlockSpec((block_s, d_model), lambda s, b: (s, 0)),
        ],
        out_specs=pl.BlockSpec((1, block_s, d_model), lambda s, b: (b, s, 0)),
        out_shape=jax.ShapeDtypeStruct(x.shape, x.dtype),
    )(x, pe)


# speedup vs baseline: 3.3228x; 1.0461x over previous
import jax
import jax.numpy as jnp
from jax.experimental import pallas as pl


def _add_pe_body(x_ref, pe_ref, o_ref):
    o_ref[...] = x_ref[...] + pe_ref[...][None]


def kernel(x, pe):
    batch, seq_len, d_model = x.shape
    block_s = 2048
    while seq_len % block_s:
        block_s //= 2
    grid = (seq_len // block_s, batch)
    return pl.pallas_call(
        _add_pe_body,
        grid=grid,
        in_specs=[
            pl.BlockSpec((1, block_s, d_model), lambda s, b: (b, s, 0)),
            pl.BlockSpec((block_s, d_model), lambda s, b: (s, 0)),
        ],
        out_specs=pl.BlockSpec((1, block_s, d_model), lambda s, b: (b, s, 0)),
        out_shape=jax.ShapeDtypeStruct(x.shape, x.dtype),
    )(x, pe)
